# Initial kernel scaffold; baseline (speedup 1.0000x reference)
import jax
import jax.numpy as jnp
from jax.experimental import pallas as pl


def kernel(x, edge_index, edge_weight):
    raise NotImplementedError("write your pallas kernel here")


# speedup vs baseline: 2.2883x; 2.2883x over previous
import functools

import jax
import jax.numpy as jnp
from jax import lax
from jax.experimental import pallas as pl
from jax.experimental.pallas import tpu as pltpu
from jax.experimental.pallas import tpu_sc as plsc

N_NODES = 10000
N_EDGES = 320000
D_FEAT = 128
K_HOPS = 2
LANES = 16

NUM_CORES = 2
NUM_SUBCORES = 16
NUM_WORKERS = NUM_CORES * NUM_SUBCORES
F_PER_TILE = D_FEAT // NUM_WORKERS

CHUNK = 4000
N_CHUNKS = N_EDGES // CHUNK
N_GROUPS = CHUNK // LANES
N_ZERO = N_NODES // LANES


SLICE = F_PER_TILE * N_NODES
N_ZERO_FLAT = SLICE // LANES


def _gcn_body(x_t, src_h, dst_h, w_h, out, tab_a, tab_b, srcb, dstb, wb):
    wid = lax.axis_index("s") * NUM_CORES + lax.axis_index("c")
    fbase = wid * SLICE

    pltpu.sync_copy(x_t.at[pl.ds(fbase, SLICE)], tab_a)

    zero16 = jnp.zeros((LANES,), jnp.float32)

    def zero_ref(ref):
        def zbody(i, _):
            ref[pl.ds(i * LANES, LANES)] = zero16
            return 0
        lax.fori_loop(0, N_ZERO_FLAT, zbody, 0)

    def hop(table, acc):
        def chunk_body(c, _):
            base = c * CHUNK
            pltpu.sync_copy(src_h.at[pl.ds(base, CHUNK)], srcb)
            pltpu.sync_copy(dst_h.at[pl.ds(base, CHUNK)], dstb)
            pltpu.sync_copy(w_h.at[pl.ds(base, CHUNK)], wb)

            def grp(i, _):
                gb = i * LANES
                s16 = srcb[pl.ds(gb, LANES)]
                d16 = dstb[pl.ds(gb, LANES)]
                w16 = wb[pl.ds(gb, LANES)]
                for f in range(F_PER_TILE):
                    off = f * N_NODES
                    g = plsc.load_gather(table, [s16 + off])
                    plsc.addupdate_scatter(acc, [d16 + off], g * w16)
                return 0

            lax.fori_loop(0, N_GROUPS, grp, 0)
            return 0

        lax.fori_loop(0, N_CHUNKS, chunk_body, 0)

    zero_ref(tab_b)
    hop(tab_a, tab_b)
    zero_ref(tab_a)
    hop(tab_b, tab_a)

    pltpu.sync_copy(tab_a, out.at[pl.ds(fbase, SLICE)])


_gcn = functools.partial(
    pl.kernel,
    out_type=jax.ShapeDtypeStruct((D_FEAT * N_NODES,), jnp.float32),
    mesh=plsc.VectorSubcoreMesh(core_axis_name="c", subcore_axis_name="s"),
    compiler_params=pltpu.CompilerParams(needs_layout_passes=False),
    scratch_types=[
        pltpu.VMEM((SLICE,), jnp.float32),
        pltpu.VMEM((SLICE,), jnp.float32),
        pltpu.VMEM((CHUNK,), jnp.int32),
        pltpu.VMEM((CHUNK,), jnp.int32),
        pltpu.VMEM((CHUNK,), jnp.float32),
    ],
)(_gcn_body)


def kernel(x, edge_index, edge_weight):
    x_t = x.T.reshape(-1)
    out_t = _gcn(x_t, edge_index[0], edge_index[1], edge_weight)
    return out_t.reshape(D_FEAT, N_NODES).T


# speedup vs baseline: 7.7658x; 3.3938x over previous
import functools

import jax
import jax.numpy as jnp
from jax import lax
from jax.experimental import pallas as pl
from jax.experimental.pallas import tpu as pltpu
from jax.experimental.pallas import tpu_sc as plsc

N_NODES = 10000
N_EDGES = 320000
D_FEAT = 128
K_HOPS = 2
LANES = 16

NUM_CORES = 2
NUM_SUBCORES = 16
NUM_WORKERS = NUM_CORES * NUM_SUBCORES
F_PER_TILE = D_FEAT // NUM_WORKERS

CHUNK = 4000
N_CHUNKS = N_EDGES // CHUNK
N_PAIRS = N_CHUNKS // 2
N_GROUPS = CHUNK // LANES
ROW = 3 * CHUNK

SLICE = F_PER_TILE * N_NODES
N_ZERO_FLAT = SLICE // LANES


def _gcn_body(x_t, ep_h, out, tab_a, tab_b, eb0, eb1, s0, s1):
    wid = lax.axis_index("s") * NUM_CORES + lax.axis_index("c")
    fbase = wid * SLICE

    pltpu.sync_copy(x_t.at[pl.ds(fbase, SLICE)], tab_a)

    zero16 = jnp.zeros((LANES,), jnp.float32)

    def zero_ref(ref):
        @plsc.parallel_loop(0, N_ZERO_FLAT, 1, unroll=8)
        def zbody(i):
            ref[pl.ds(i * LANES, LANES)] = zero16

    def compute(eb, table, acc):
        @plsc.parallel_loop(0, N_GROUPS, 1, unroll=4)
        def grp(i):
            gb = i * LANES
            s16 = eb[pl.ds(gb, LANES)]
            d16 = eb[pl.ds(CHUNK + gb, LANES)]
            w16 = plsc.bitcast(eb[pl.ds(2 * CHUNK + gb, LANES)], jnp.float32)
            for f in range(F_PER_TILE):
                off = f * N_NODES
                g = plsc.load_gather(table, [s16 + off])
                plsc.addupdate_scatter(acc, [d16 + off], g * w16)

    def hop(table, acc):
        pltpu.async_copy(ep_h.at[0], eb0, s0)
        pltpu.async_copy(ep_h.at[1], eb1, s1)

        def pair(p, _):
            c = 2 * p
            pltpu.make_async_copy(ep_h.at[0], eb0, s0).wait()
            compute(eb0, table, acc)
            pltpu.async_copy(ep_h.at[c + 2], eb0, s0)
            pltpu.make_async_copy(ep_h.at[0], eb1, s1).wait()
            compute(eb1, table, acc)
            pltpu.async_copy(ep_h.at[c + 3], eb1, s1)
            return 0

        lax.fori_loop(0, N_PAIRS, pair, 0)
        pltpu.make_async_copy(ep_h.at[0], eb0, s0).wait()
        pltpu.make_async_copy(ep_h.at[0], eb1, s1).wait()

    zero_ref(tab_b)
    hop(tab_a, tab_b)
    zero_ref(tab_a)
    hop(tab_b, tab_a)

    pltpu.sync_copy(tab_a, out.at[pl.ds(fbase, SLICE)])


_gcn = functools.partial(
    pl.kernel,
    out_type=jax.ShapeDtypeStruct((D_FEAT * N_NODES,), jnp.float32),
    mesh=plsc.VectorSubcoreMesh(core_axis_name="c", subcore_axis_name="s"),
    compiler_params=pltpu.CompilerParams(needs_layout_passes=False),
    scratch_types=[
        pltpu.VMEM((SLICE,), jnp.float32),
        pltpu.VMEM((SLICE,), jnp.float32),
        pltpu.VMEM((ROW,), jnp.int32),
        pltpu.VMEM((ROW,), jnp.int32),
        pltpu.SemaphoreType.DMA,
        pltpu.SemaphoreType.DMA,
    ],
)(_gcn_body)


def kernel(x, edge_index, edge_weight):
    x_t = x.T.reshape(-1)
    wbits = lax.bitcast_convert_type(edge_weight, jnp.int32)
    e3 = jnp.stack([edge_index[0], edge_index[1], wbits])
    epack = (e3.reshape(3, N_CHUNKS, CHUNK)
             .transpose(1, 0, 2)
             .reshape(N_CHUNKS, ROW))
    epack = jnp.concatenate(
        [epack, jnp.zeros((2, ROW), jnp.int32)], axis=0)
    out_t = _gcn(x_t, epack)
    return out_t.reshape(D_FEAT, N_NODES).T
